# baseline (speedup 1.0000x reference)
import functools

import jax
import jax.numpy as jnp
import numpy as np
from jax import lax
from jax.experimental import pallas as pl
from jax.experimental.pallas import tpu as pltpu
from jax.experimental.pallas import tpu_sc as plsc

EMBED = 2048
FEATS = 32768
K = 64
B = 4096
CHUNK = 16
NCHUNK = FEATS // CHUNK
PS = 16
NSUB = 8

EBB = 256
EFB = 2048
DBB = 256
DFB = 2048


def _enc_body(x_ref, w_ref, p_ref, cm_ref):
    x = x_ref[...]
    w = w_ref[...]
    p = jax.lax.dot_general(x, w, (((1,), (1,)), ((), ())),
                            preferred_element_type=jnp.float32)
    p_ref[...] = p
    m = p
    for sh in (1, 2, 4, 8):
        m = jnp.maximum(m, pltpu.roll(m, EFB - sh, 1))
    lane = jax.lax.broadcasted_iota(jnp.int32, (EFB, EFB // CHUNK), 0)
    col = jax.lax.broadcasted_iota(jnp.int32, (EFB, EFB // CHUNK), 1)
    sel = (lane == col * CHUNK).astype(jnp.float32)
    cm_ref[...] = jax.lax.dot_general(
        m, sel, (((1,), (0,)), ((), ())), preferred_element_type=jnp.float32,
        precision=jax.lax.Precision.HIGHEST)


def _encoder(embed0, enc_weight, nb):
    grid = (nb // EBB, FEATS // EFB)
    return pl.pallas_call(
        _enc_body,
        grid=grid,
        in_specs=[
            pl.BlockSpec((EBB, EMBED), lambda i, j: (i, 0)),
            pl.BlockSpec((EFB, EMBED), lambda i, j: (j, 0)),
        ],
        out_specs=[
            pl.BlockSpec((EBB, EFB), lambda i, j: (i, j)),
            pl.BlockSpec((EBB, EFB // CHUNK), lambda i, j: (i, j)),
        ],
        out_shape=[
            jax.ShapeDtypeStruct((nb, FEATS), jnp.float32),
            jax.ShapeDtypeStruct((nb, NCHUNK), jnp.float32),
        ],
    )(embed0, enc_weight)


def _dec_body(p_ref, v_ref, m_ref, lk_ref, b_ref, out_ref, ent_ref,
              acc, s0, s1):
    k = pl.program_id(1)
    nk = pl.num_programs(1)

    @pl.when(k == 0)
    def _():
        acc[...] = jnp.zeros_like(acc)
        s0[...] = jnp.zeros_like(s0)
        s1[...] = jnp.zeros_like(s1)

    p = p_ref[...]
    v = v_ref[...][:, 0:1]
    mx = m_ref[...][:, 0:1]
    mask = p >= v
    pm = jnp.where(mask, p, 0.0)
    acc[...] += jax.lax.dot_general(pm, lk_ref[...], (((1,), (0,)), ((), ())),
                                    preferred_element_type=jnp.float32)
    e = jnp.where(mask, jnp.exp(p - mx), 0.0)
    s0[...] += jnp.sum(e, axis=1, keepdims=True)
    s1[...] += jnp.sum(p * e, axis=1, keepdims=True)

    @pl.when(k == nk - 1)
    def _():
        recon = acc[...] + b_ref[0:1, :]
        nrm = jnp.sqrt(jnp.sum(recon * recon, axis=1, keepdims=True))
        out_ref[...] = recon / jnp.maximum(nrm, 1e-12)
        a0 = s0[...][:, 0:1]
        a1 = s1[...][:, 0:1]
        ent = (mx + jnp.log(a0)) - a1 / a0
        ent_ref[...] = jnp.broadcast_to(ent, ent_ref.shape)


def _decoder(project, v64t, rowmaxt, lookup, bias_t, nb):
    grid = (nb // DBB, FEATS // DFB)
    return pl.pallas_call(
        _dec_body,
        grid=grid,
        in_specs=[
            pl.BlockSpec((DBB, DFB), lambda i, k: (i, k)),
            pl.BlockSpec((DBB, 128), lambda i, k: (i, 0)),
            pl.BlockSpec((DBB, 128), lambda i, k: (i, 0)),
            pl.BlockSpec((DFB, EMBED), lambda i, k: (k, 0)),
            pl.BlockSpec((8, EMBED), lambda i, k: (0, 0)),
        ],
        out_specs=[
            pl.BlockSpec((DBB, EMBED), lambda i, k: (i, 0)),
            pl.BlockSpec((DBB, 128), lambda i, k: (i, 0)),
        ],
        out_shape=[
            jax.ShapeDtypeStruct((nb, EMBED), jnp.float32),
            jax.ShapeDtypeStruct((nb, 128), jnp.float32),
        ],
        scratch_shapes=[
            pltpu.VMEM((DBB, EMBED), jnp.float32),
            pltpu.VMEM((DBB, 128), jnp.float32),
            pltpu.VMEM((DBB, 128), jnp.float32),
        ],
    )(project, v64t, rowmaxt, lookup, bias_t)


def _to_mu(v):
    b = plsc.bitcast(v, jnp.int32)
    flip = jnp.where(b < 0, np.int32(-1), np.int32(-2147483648))
    return plsc.bitcast(b ^ flip, jnp.uint32)


def _from_mu(mu):
    y = plsc.bitcast(mu, jnp.int32)
    flip = jnp.where(y >= 0, np.int32(-1), np.int32(-2147483648))
    return plsc.bitcast(y ^ flip, jnp.float32)


def _v16(x):
    return jnp.zeros((16,), jnp.int32) + x


def _radix_select(load, npos, rank0, hist, lane):
    pref = jnp.zeros((16,), jnp.uint32)
    rank = rank0
    ones = jnp.ones((16,), jnp.int32)
    zeros = jnp.zeros((16,), jnp.int32)
    for r in range(4):
        shift = np.uint32(24 - 8 * r)
        hi_shift = np.uint32(32 - 8 * r)

        def _clr(j, _):
            hist[pl.ds(j * 16, 16)] = zeros
            return 0
        lax.fori_loop(0, 256 * NSUB, _clr, 0, unroll=8)

        def _hbody(pos8, _, r=r, shift=shift, hi_shift=hi_shift, pref=pref):
            for u in range(NSUB):
                pos = pos8 * NSUB + u
                mu = _to_mu(load(pos))
                bin_i = ((mu >> shift) & np.uint32(255)).astype(jnp.int32)
                addr = bin_i * 16 + lane + u * 4096
                if r == 0:
                    plsc.addupdate_scatter(hist, [addr], ones)
                else:
                    m = (mu >> hi_shift) == pref
                    plsc.addupdate_scatter(hist, [addr], ones, mask=m)
            return 0
        lax.fori_loop(0, npos // NSUB, _hbody, 0)

        def _sbody(i, st):
            acc, found, binb, cgt = st
            b = 255 - i
            h = hist[pl.ds(b * 16, 16)]
            for u in range(1, NSUB):
                h = h + hist[pl.ds(b * 16 + u * 4096, 16)]
            acc2 = acc + h
            hit = jnp.logical_and(jnp.logical_not(found), acc2 >= rank)
            binb = jnp.where(hit, b, binb)
            cgt = jnp.where(hit, acc, cgt)
            return acc2, jnp.logical_or(found, hit), binb, cgt

        false16 = jnp.zeros((16,), jnp.bool_)
        _, _, binb, cgt = lax.fori_loop(0, 256, _sbody,
                                        (zeros, false16, zeros, zeros),
                                        unroll=4)
        rank = rank - cgt
        pref = (pref << np.uint32(8)) | binb.astype(jnp.uint32)
    return pref


def _sc_body(nbw, cm_hbm, prv_hbm, v64_hbm,
             cm_stage, cm_pad, cand_idx, cand_pad, bufa, bufb, hist, outv,
             sem):
    wid = lax.axis_index("s") * 2 + lax.axis_index("c")
    lane = lax.iota(jnp.int32, 16)
    k64 = _v16(K)

    def group(g, _):
        base = wid * nbw + g * 16
        cps = [pltpu.async_copy(cm_hbm.at[base + r], cm_stage.at[r], sem)
               for r in range(16)]
        for c in cps:
            c.wait()

        for r in range(16):
            def _tb(j, _, r=r):
                v = cm_stage[r, pl.ds(j * 16, 16)]
                plsc.store_scatter(cm_pad, [(j * 16 + lane) * PS + r], v)
                return 0
            lax.fori_loop(0, NCHUNK // 16, _tb, 0, unroll=8)

        def load_cm(pos):
            return cm_pad[pl.ds(pos * PS, 16)]

        mu_t = _radix_select(load_cm, NCHUNK, k64, hist, lane)
        t = _from_mu(mu_t)

        rowbase = (base + lane) * NCHUNK

        def _cbody(pos, cnt):
            v = load_cm(pos)
            m = v >= t
            take = jnp.logical_and(m, cnt < K)
            slot = jnp.minimum(cnt, K - 1)
            plsc.store_scatter(cand_idx, [lane * K + slot], rowbase + pos,
                               mask=take)
            return cnt + m.astype(jnp.int32)
        lax.fori_loop(0, NCHUNK, _cbody, jnp.zeros((16,), jnp.int32), unroll=8)

        bufs = (bufa, bufb)
        cp = pltpu.async_copy(prv_hbm.at[cand_idx.at[pl.ds(0, K)]], bufa, sem)
        for r in range(16):
            cp.wait()
            if r < 15:
                nxt = pltpu.async_copy(
                    prv_hbm.at[cand_idx.at[pl.ds((r + 1) * K, K)]],
                    bufs[(r + 1) % 2], sem)
            cur = bufs[r % 2]

            def _eb(c, _, r=r, cur=cur):
                v = cur[c]
                plsc.store_scatter(cand_pad, [(c * 16 + lane) * PS + r], v)
                return 0
            lax.fori_loop(0, K, _eb, 0, unroll=8)
            if r < 15:
                cp = nxt

        def load_el(pos):
            return cand_pad[pl.ds(pos * PS, 16)]

        mu_v = _radix_select(load_el, K * CHUNK, k64, hist, lane)
        outv[pl.ds(g * 16, 16)] = _from_mu(mu_v)
        return 0

    lax.fori_loop(0, nbw // 16, group, 0)
    pltpu.sync_copy(outv, v64_hbm.at[pl.ds(wid * nbw, nbw)])


def _sc_select(chunkmax, project_view, nb):
    nbw = nb // 32
    mesh = plsc.VectorSubcoreMesh(core_axis_name="c", subcore_axis_name="s")
    f = functools.partial(
        pl.kernel, mesh=mesh,
        compiler_params=pltpu.CompilerParams(needs_layout_passes=False,
                                             use_tc_tiling_on_sc=False),
        out_type=jax.ShapeDtypeStruct((nb,), jnp.float32),
        scratch_types=[
            pltpu.VMEM((16, NCHUNK), jnp.float32),
            pltpu.VMEM((NCHUNK * PS,), jnp.float32),
            pltpu.VMEM((16 * K,), jnp.int32),
            pltpu.VMEM((K * CHUNK * PS,), jnp.float32),
            pltpu.VMEM((K, CHUNK), jnp.float32),
            pltpu.VMEM((K, CHUNK), jnp.float32),
            pltpu.VMEM((256 * 16 * NSUB,), jnp.int32),
            pltpu.VMEM((nbw,), jnp.float32),
            pltpu.SemaphoreType.DMA,
        ],
    )
    def body(cm_hbm, prv_hbm, v64_hbm, cm_stage, cm_pad, cand_idx,
             cand_pad, bufa, bufb, hist, outv, sem):
        _sc_body(nbw, cm_hbm, prv_hbm, v64_hbm, cm_stage, cm_pad, cand_idx,
                 cand_pad, bufa, bufb, hist, outv, sem)
    return f(body)(chunkmax, project_view)


def kernel(embed, bias, enc_weight, lookup):
    embed0 = embed - bias
    bias_t = jnp.broadcast_to(bias[None, :], (8, EMBED))
    nh = 4
    nb = B // nh
    outs = []
    for h in range(nh):
        e0 = jax.lax.slice_in_dim(embed0, h * nb, (h + 1) * nb, axis=0)
        project, chunkmax = _encoder(e0, enc_weight, nb)
        v64 = _sc_select(chunkmax, project.reshape(nb * NCHUNK, CHUNK), nb)
        rowmax = jnp.max(chunkmax, axis=1)
        v64t = jnp.broadcast_to(v64[:, None], (nb, 128))
        rowmaxt = jnp.broadcast_to(rowmax[:, None], (nb, 128))
        outs.append(_decoder(project, v64t, rowmaxt, lookup, bias_t, nb))
    embed1 = jnp.concatenate([o[0] for o in outs], axis=0)
    ent = jnp.concatenate([o[1][:, 0] for o in outs], axis=0)
    return (embed1, ent)


# speedup vs baseline: 1.0494x; 1.0494x over previous
import functools

import jax
import jax.numpy as jnp
import numpy as np
from jax import lax
from jax.experimental import pallas as pl
from jax.experimental.pallas import tpu as pltpu
from jax.experimental.pallas import tpu_sc as plsc

EMBED = 2048
FEATS = 32768
K = 64
B = 4096
CHUNK = 16
NCHUNK = FEATS // CHUNK
PS = 16

EBB = 256
EFB = 2048
DBB = 256
DFB = 2048


def _enc_body(x_ref, w_ref, p_ref, cm_ref):
    x = x_ref[...]
    w = w_ref[...]
    p = jax.lax.dot_general(x, w, (((1,), (1,)), ((), ())),
                            preferred_element_type=jnp.float32)
    p_ref[...] = p
    m = p
    for sh in (1, 2, 4, 8):
        m = jnp.maximum(m, pltpu.roll(m, EFB - sh, 1))
    lane = jax.lax.broadcasted_iota(jnp.int32, (EFB, EFB // CHUNK), 0)
    col = jax.lax.broadcasted_iota(jnp.int32, (EFB, EFB // CHUNK), 1)
    sel = (lane == col * CHUNK).astype(jnp.float32)
    cm_ref[...] = jax.lax.dot_general(
        m, sel, (((1,), (0,)), ((), ())), preferred_element_type=jnp.float32,
        precision=jax.lax.Precision.HIGHEST)


def _encoder(embed0, enc_weight, nb):
    grid = (nb // EBB, FEATS // EFB)
    return pl.pallas_call(
        _enc_body,
        grid=grid,
        in_specs=[
            pl.BlockSpec((EBB, EMBED), lambda i, j: (i, 0)),
            pl.BlockSpec((EFB, EMBED), lambda i, j: (j, 0)),
        ],
        out_specs=[
            pl.BlockSpec((EBB, EFB), lambda i, j: (i, j)),
            pl.BlockSpec((EBB, EFB // CHUNK), lambda i, j: (i, j)),
        ],
        out_shape=[
            jax.ShapeDtypeStruct((nb, FEATS), jnp.float32),
            jax.ShapeDtypeStruct((nb, NCHUNK), jnp.float32),
        ],
    )(embed0, enc_weight)


def _dec_body(p_ref, v_ref, m_ref, lk_ref, b_ref, out_ref, ent_ref,
              acc, s0, s1):
    k = pl.program_id(1)
    nk = pl.num_programs(1)

    @pl.when(k == 0)
    def _():
        acc[...] = jnp.zeros_like(acc)
        s0[...] = jnp.zeros_like(s0)
        s1[...] = jnp.zeros_like(s1)

    p = p_ref[...]
    v = v_ref[...][:, 0:1]
    mx = m_ref[...][:, 0:1]
    mask = p >= v
    pm = jnp.where(mask, p, 0.0)
    acc[...] += jax.lax.dot_general(pm, lk_ref[...], (((1,), (0,)), ((), ())),
                                    preferred_element_type=jnp.float32)
    e = jnp.where(mask, jnp.exp(p - mx), 0.0)
    s0[...] += jnp.sum(e, axis=1, keepdims=True)
    s1[...] += jnp.sum(p * e, axis=1, keepdims=True)

    @pl.when(k == nk - 1)
    def _():
        recon = acc[...] + b_ref[0:1, :]
        nrm = jnp.sqrt(jnp.sum(recon * recon, axis=1, keepdims=True))
        out_ref[...] = recon / jnp.maximum(nrm, 1e-12)
        a0 = s0[...][:, 0:1]
        a1 = s1[...][:, 0:1]
        ent = (mx + jnp.log(a0)) - a1 / a0
        ent_ref[...] = jnp.broadcast_to(ent, ent_ref.shape)


def _decoder(project, v64t, rowmaxt, lookup, bias_t, nb):
    grid = (nb // DBB, FEATS // DFB)
    return pl.pallas_call(
        _dec_body,
        grid=grid,
        in_specs=[
            pl.BlockSpec((DBB, DFB), lambda i, k: (i, k)),
            pl.BlockSpec((DBB, 128), lambda i, k: (i, 0)),
            pl.BlockSpec((DBB, 128), lambda i, k: (i, 0)),
            pl.BlockSpec((DFB, EMBED), lambda i, k: (k, 0)),
            pl.BlockSpec((8, EMBED), lambda i, k: (0, 0)),
        ],
        out_specs=[
            pl.BlockSpec((DBB, EMBED), lambda i, k: (i, 0)),
            pl.BlockSpec((DBB, 128), lambda i, k: (i, 0)),
        ],
        out_shape=[
            jax.ShapeDtypeStruct((nb, EMBED), jnp.float32),
            jax.ShapeDtypeStruct((nb, 128), jnp.float32),
        ],
        scratch_shapes=[
            pltpu.VMEM((DBB, EMBED), jnp.float32),
            pltpu.VMEM((DBB, 128), jnp.float32),
            pltpu.VMEM((DBB, 128), jnp.float32),
        ],
    )(project, v64t, rowmaxt, lookup, bias_t)


def _to_mu(v):
    b = plsc.bitcast(v, jnp.int32)
    flip = jnp.where(b < 0, np.int32(-1), np.int32(-2147483648))
    return plsc.bitcast(b ^ flip, jnp.uint32)


def _from_mu(mu):
    y = plsc.bitcast(mu, jnp.int32)
    flip = jnp.where(y >= 0, np.int32(-1), np.int32(-2147483648))
    return plsc.bitcast(y ^ flip, jnp.float32)


def _v16(x):
    return jnp.zeros((16,), jnp.int32) + x


def _radix_select(load, npos, rank0, hist, lane):
    pref = jnp.zeros((16,), jnp.uint32)
    rank = rank0
    ones = jnp.ones((16,), jnp.int32)
    zeros = jnp.zeros((16,), jnp.int32)
    for r in range(4):
        shift = np.uint32(24 - 8 * r)
        hi_shift = np.uint32(32 - 8 * r)

        def _clr(j, _):
            hist[pl.ds(j * 16, 16)] = zeros
            return 0
        lax.fori_loop(0, 256, _clr, 0, unroll=4)

        def _hbody(pos, _, r=r, shift=shift, hi_shift=hi_shift, pref=pref):
            mu = _to_mu(load(pos))
            bin_i = ((mu >> shift) & np.uint32(255)).astype(jnp.int32)
            if r == 0:
                plsc.addupdate_scatter(hist, [bin_i * 16 + lane], ones)
            else:
                m = (mu >> hi_shift) == pref
                plsc.addupdate_scatter(hist, [bin_i * 16 + lane], ones, mask=m)
            return 0
        lax.fori_loop(0, npos, _hbody, 0, unroll=8)

        def _sbody(i, st):
            acc, found, binb, cgt = st
            b = 255 - i
            h = hist[pl.ds(b * 16, 16)]
            acc2 = acc + h
            hit = jnp.logical_and(jnp.logical_not(found), acc2 >= rank)
            binb = jnp.where(hit, b, binb)
            cgt = jnp.where(hit, acc, cgt)
            return acc2, jnp.logical_or(found, hit), binb, cgt

        false16 = jnp.zeros((16,), jnp.bool_)
        _, _, binb, cgt = lax.fori_loop(0, 256, _sbody,
                                        (zeros, false16, zeros, zeros),
                                        unroll=4)
        rank = rank - cgt
        pref = (pref << np.uint32(8)) | binb.astype(jnp.uint32)
    return pref


def _sc_body(nbw, cm_hbm, prv_hbm, v64_hbm,
             cm_stage, cm_pad, cand_idx, cand_pad, bufa, bufb, hist, outv,
             sem):
    wid = lax.axis_index("s") * 2 + lax.axis_index("c")
    lane = lax.iota(jnp.int32, 16)
    k64 = _v16(K)

    def group(g, _):
        base = wid * nbw + g * 16
        cps = [pltpu.async_copy(cm_hbm.at[base + r], cm_stage.at[r], sem)
               for r in range(16)]
        for c in cps:
            c.wait()

        for r in range(16):
            def _tb(j, _, r=r):
                v = cm_stage[r, pl.ds(j * 16, 16)]
                plsc.store_scatter(cm_pad, [(j * 16 + lane) * PS + r], v)
                return 0
            lax.fori_loop(0, NCHUNK // 16, _tb, 0, unroll=8)

        def load_cm(pos):
            return cm_pad[pl.ds(pos * PS, 16)]

        mu_t = _radix_select(load_cm, NCHUNK, k64, hist, lane)
        t = _from_mu(mu_t)

        rowbase = (base + lane) * NCHUNK

        def _cbody(pos, cnt):
            v = load_cm(pos)
            m = v >= t
            take = jnp.logical_and(m, cnt < K)
            slot = jnp.minimum(cnt, K - 1)
            plsc.store_scatter(cand_idx, [lane * K + slot], rowbase + pos,
                               mask=take)
            return cnt + m.astype(jnp.int32)
        lax.fori_loop(0, NCHUNK, _cbody, jnp.zeros((16,), jnp.int32), unroll=8)

        bufs = (bufa, bufb)
        cp = pltpu.async_copy(prv_hbm.at[cand_idx.at[pl.ds(0, K)]], bufa, sem)
        for r in range(16):
            cp.wait()
            if r < 15:
                nxt = pltpu.async_copy(
                    prv_hbm.at[cand_idx.at[pl.ds((r + 1) * K, K)]],
                    bufs[(r + 1) % 2], sem)
            cur = bufs[r % 2]

            def _eb(c, _, r=r, cur=cur):
                v = cur[c]
                plsc.store_scatter(cand_pad, [(c * 16 + lane) * PS + r], v)
                return 0
            lax.fori_loop(0, K, _eb, 0, unroll=8)
            if r < 15:
                cp = nxt

        def load_el(pos):
            return cand_pad[pl.ds(pos * PS, 16)]

        mu_v = _radix_select(load_el, K * CHUNK, k64, hist, lane)
        outv[pl.ds(g * 16, 16)] = _from_mu(mu_v)
        return 0

    lax.fori_loop(0, nbw // 16, group, 0)
    pltpu.sync_copy(outv, v64_hbm.at[pl.ds(wid * nbw, nbw)])


def _sc_select(chunkmax, project_view, nb):
    nbw = nb // 32
    mesh = plsc.VectorSubcoreMesh(core_axis_name="c", subcore_axis_name="s")
    f = functools.partial(
        pl.kernel, mesh=mesh,
        compiler_params=pltpu.CompilerParams(needs_layout_passes=False,
                                             use_tc_tiling_on_sc=False),
        out_type=jax.ShapeDtypeStruct((nb,), jnp.float32),
        scratch_types=[
            pltpu.VMEM((16, NCHUNK), jnp.float32),
            pltpu.VMEM((NCHUNK * PS,), jnp.float32),
            pltpu.VMEM((16 * K,), jnp.int32),
            pltpu.VMEM((K * CHUNK * PS,), jnp.float32),
            pltpu.VMEM((K, CHUNK), jnp.float32),
            pltpu.VMEM((K, CHUNK), jnp.float32),
            pltpu.VMEM((256 * 16,), jnp.int32),
            pltpu.VMEM((nbw,), jnp.float32),
            pltpu.SemaphoreType.DMA,
        ],
    )
    def body(cm_hbm, prv_hbm, v64_hbm, cm_stage, cm_pad, cand_idx,
             cand_pad, bufa, bufb, hist, outv, sem):
        _sc_body(nbw, cm_hbm, prv_hbm, v64_hbm, cm_stage, cm_pad, cand_idx,
                 cand_pad, bufa, bufb, hist, outv, sem)
    return f(body)(chunkmax, project_view)


def kernel(embed, bias, enc_weight, lookup):
    embed0 = embed - bias
    bias_t = jnp.broadcast_to(bias[None, :], (8, EMBED))
    nh = 8
    nb = B // nh
    outs = []
    for h in range(nh):
        e0 = jax.lax.slice_in_dim(embed0, h * nb, (h + 1) * nb, axis=0)
        project, chunkmax = _encoder(e0, enc_weight, nb)
        v64 = _sc_select(chunkmax, project.reshape(nb * NCHUNK, CHUNK), nb)
        rowmax = jnp.max(chunkmax, axis=1)
        v64t = jnp.broadcast_to(v64[:, None], (nb, 128))
        rowmaxt = jnp.broadcast_to(rowmax[:, None], (nb, 128))
        outs.append(_decoder(project, v64t, rowmaxt, lookup, bias_t, nb))
    embed1 = jnp.concatenate([o[0] for o in outs], axis=0)
    ent = jnp.concatenate([o[1][:, 0] for o in outs], axis=0)
    return (embed1, ent)
